# baseline (speedup 1.0000x reference)
import functools

import jax
import jax.numpy as jnp
from jax import lax
from jax.experimental import pallas as pl
from jax.experimental.pallas import tpu as pltpu
from jax.experimental.pallas import tpu_sc as plsc

_NUM_CORES = 2
_NUM_SUBCORES = 16
_LANES = 16
_NW = _NUM_CORES * _NUM_SUBCORES

_OUT_CHUNK = 4096
_NBUF = 2


@functools.lru_cache(maxsize=None)
def _build(num_fields, vocab_rows, dim, batch):
    n_rows = num_fields * dim
    per_w = n_rows // _NW
    n_chunks = batch // _OUT_CHUNK
    mesh = plsc.VectorSubcoreMesh(core_axis_name="c", subcore_axis_name="s")

    @functools.partial(
        pl.kernel,
        out_type=jax.ShapeDtypeStruct((n_rows, batch), jnp.float32),
        mesh=mesh,
        compiler_params=pltpu.CompilerParams(
            use_tc_tiling_on_sc=True, needs_layout_passes=False
        ),
        scratch_types=[
            pltpu.MemorySpace.VMEM((vocab_rows,), jnp.float32),
            pltpu.MemorySpace.VMEM((batch,), jnp.int32),
            pltpu.MemorySpace.VMEM((_NBUF, _OUT_CHUNK), jnp.float32),
            pltpu.SemaphoreType.DMA,
        ],
    )
    def run(idx_hbm, tab_hbm, out_hbm, tab_v, idx_v, out_v, sem):
        wid = lax.axis_index("s") * _NUM_CORES + lax.axis_index("c")
        r0 = wid * per_w

        def task_body(i, prev_f):
            r = r0 + i
            f = r // dim

            @pl.when(f != prev_f)
            def _():
                pltpu.sync_copy(idx_hbm.at[f], idx_v)

            pltpu.sync_copy(tab_hbm.at[r], tab_v)

            for h in range(n_chunks):
                b = h % _NBUF
                base = h * _OUT_CHUNK

                def drain():
                    pltpu.make_async_copy(
                        out_v.at[0], out_hbm.at[r0, pl.ds(0, _OUT_CHUNK)], sem
                    ).wait()

                if h < _NBUF:
                    @pl.when(i > 0)
                    def _():
                        drain()
                else:
                    drain()

                def vec_body(j, base=base, b=b):
                    off = j * _LANES
                    iv = idx_v[pl.ds(base + off, _LANES)]
                    out_v[b, pl.ds(off, _LANES)] = plsc.load_gather(tab_v, [iv])

                plsc.parallel_loop(0, _OUT_CHUNK // _LANES, unroll=16)(vec_body)
                pltpu.async_copy(
                    out_v.at[b], out_hbm.at[r, pl.ds(base, _OUT_CHUNK)], sem
                )
            return f

        lax.fori_loop(0, per_w, task_body, -1)

        for _ in range(min(_NBUF, n_chunks)):
            pltpu.make_async_copy(
                out_v.at[0], out_hbm.at[r0, pl.ds(0, _OUT_CHUNK)], sem
            ).wait()

    return run


def kernel(inputs, tables):
    batch, num_fields = inputs.shape
    _, vocab_rows, dim = tables.shape
    idx_t = inputs.T.astype(jnp.int32)
    tab2 = jnp.transpose(tables, (0, 2, 1)).reshape(
        num_fields * dim, vocab_rows
    )
    out2 = _build(num_fields, vocab_rows, dim, batch)(idx_t, tab2)
    out3 = out2.reshape(num_fields, dim, batch)
    return jnp.transpose(out3, (2, 0, 1))
